# baseline (speedup 1.0000x reference)
import functools

import jax
import jax.numpy as jnp
from jax import lax
from jax.experimental import pallas as pl
from jax.experimental.pallas import tpu as pltpu
from jax.experimental.pallas import tpu_sc as plsc

VIGILANCE = 0.75
SUB = 8
ROWS_PER_PROG = 128
LANES = 128
SC_ROWS = 128
SC_NW = 32
SC_L = 16


def _combine(s1, j1, s2, j2):
    take2 = (s2 > s1) | ((s2 == s1) & (j2 < j1))
    return jnp.maximum(s1, s2), jnp.where(take2, j2, j1)


def _tc_body(x_ref, c_ref, out_ref, idx_ref, ctb_ref, ps_ref, pj_ref):
    d_dim = x_ref.shape[1]
    k_dim = c_ref.shape[0]
    nsub = ROWS_PER_PROG // SUB
    nchunk = k_dim // LANES

    @pl.when(pl.program_id(0) == 0)
    def _fill():
        ct = jnp.transpose(c_ref[...], (1, 0))
        for d in range(d_dim):
            ctb_ref[d] = jnp.broadcast_to(ct[d:d + 1, :], (SUB, k_dim))

    def sub_block(jb):
        base = jb * SUB
        x = x_ref[pl.ds(base, SUB), :]
        den = jnp.sum(x, axis=1, keepdims=True)
        acc = jnp.zeros((SUB, k_dim), jnp.float32)
        for d in range(d_dim):
            acc = acc + jnp.minimum(x[:, d:d + 1], ctb_ref[d])
        m = acc / den
        s = jnp.where(m >= VIGILANCE, m, jnp.zeros_like(m))
        out_ref[pl.ds(base, SUB), :] = s
        lane = lax.broadcasted_iota(jnp.int32, (SUB, LANES), 1)
        pairs = [(s[:, c * LANES:(c + 1) * LANES], lane + c * LANES)
                 for c in range(nchunk)]
        while len(pairs) > 1:
            nxt = []
            for a in range(0, len(pairs) - 1, 2):
                nxt.append(_combine(*pairs[a], *pairs[a + 1]))
            if len(pairs) % 2:
                nxt.append(pairs[-1])
            pairs = nxt
        ps_ref[jb] = pairs[0][0]
        pj_ref[jb] = pairs[0][1]

    def pair_iter(j, carry):
        sub_block(2 * j)
        sub_block(2 * j + 1)
        return carry

    lax.fori_loop(0, nsub // 2, pair_iter, 0)

    sv = ps_ref[...]
    jv = pj_ref[...]
    t = 1
    while t < LANES:
        sr = pltpu.roll(sv, t, 2)
        jr = pltpu.roll(jv, t, 2)
        sv, jv = _combine(sv, jv, sr, jr)
        t *= 2
    idx_ref[...] = jv[:, :, 0:1].reshape(ROWS_PER_PROG, 1)


def _tc_call(x, categories, tc_rows, b):
    d_dim = x.shape[1]
    k_dim = categories.shape[0]
    return pl.pallas_call(
        _tc_body,
        grid=(tc_rows // ROWS_PER_PROG,),
        in_specs=[
            pl.BlockSpec((ROWS_PER_PROG, d_dim), lambda i: (i, 0)),
            pl.BlockSpec((k_dim, d_dim), lambda i: (0, 0)),
        ],
        out_specs=[
            pl.BlockSpec((ROWS_PER_PROG, k_dim), lambda i: (i, 0)),
            pl.BlockSpec((ROWS_PER_PROG, 1), lambda i: (i, 0)),
        ],
        out_shape=[
            jax.ShapeDtypeStruct((b, k_dim), jnp.float32),
            jax.ShapeDtypeStruct((b, 1), jnp.int32),
        ],
        scratch_shapes=[
            pltpu.VMEM((d_dim, SUB, k_dim), jnp.float32),
            pltpu.VMEM((ROWS_PER_PROG // SUB, SUB, LANES), jnp.float32),
            pltpu.VMEM((ROWS_PER_PROG // SUB, SUB, LANES), jnp.int32),
        ],
    )(x, categories)


SC_KB = 8


def _sc_body(d_dim, k_dim, nr, row0, xs_hbm, ct_hbm, out_hbm, idxp_hbm,
             ct_v, s_v, x_v, idx_v):
    wid = lax.axis_index("s") * 2 + lax.axis_index("c")
    pltpu.sync_copy(ct_hbm, ct_v)
    pltpu.sync_copy(xs_hbm.at[pl.ds(row0 + wid * nr, nr)], x_v)
    nxc = d_dim // SC_L
    lane = lax.broadcasted_iota(jnp.int32, (SC_L,), 0)
    idxvec = jnp.zeros((SC_L,), jnp.int32)
    for i in range(nr):
        xc = [x_v[i, pl.ds(c * SC_L, SC_L)] for c in range(nxc)]
        dv = xc[0]
        for c in range(1, nxc):
            dv = dv + xc[c]
        den = dv[0]
        for l in range(1, SC_L):
            den = den + dv[l]
        denv = jnp.full((SC_L,), den, jnp.float32)
        bestv = jnp.full((SC_L,), -jnp.inf, jnp.float32)
        bestj = jnp.full((SC_L,), k_dim, jnp.int32)

        def kc_step(kcg, carry):
            bv, bj = carry
            accs = [jnp.zeros((SC_L,), jnp.float32) for _ in range(SC_KB)]
            for d in range(d_dim):
                xs = jnp.full((SC_L,), xc[d // SC_L][d % SC_L], jnp.float32)
                for bsl in range(SC_KB):
                    col = ct_v[d, pl.ds((kcg * SC_KB + bsl) * SC_L, SC_L)]
                    accs[bsl] = accs[bsl] + jnp.minimum(xs, col)
            for bsl in range(SC_KB):
                m = accs[bsl] / denv
                s = jnp.where(m >= VIGILANCE, m, jnp.zeros_like(m))
                s_v[pl.ds((kcg * SC_KB + bsl) * SC_L, SC_L)] = s
                j = lane + (kcg * SC_KB + bsl) * SC_L
                bv, bj = _combine(bv, bj, s, j)
            return bv, bj

        bestv, bestj = lax.fori_loop(0, k_dim // (SC_L * SC_KB), kc_step,
                                     (bestv, bestj))
        best = bestv[0]
        bidx = bestj[0]
        for l in range(1, SC_L):
            v = bestv[l]
            j = bestj[l]
            take = (v > best) | ((v == best) & (j < bidx))
            best = jnp.where(take, v, best)
            bidx = jnp.where(take, j, bidx)
        idxvec = jnp.where(lane == i, jnp.full((SC_L,), bidx, jnp.int32),
                           idxvec)
        pltpu.sync_copy(s_v, out_hbm.at[wid * nr + i])
    idx_v[...] = idxvec
    pltpu.sync_copy(idx_v, idxp_hbm.at[wid])


def _sc_call(x, ct, row0, n_rows):
    d_dim = x.shape[1]
    k_dim = ct.shape[1]
    nr = n_rows // SC_NW
    mesh = plsc.VectorSubcoreMesh(core_axis_name="c", subcore_axis_name="s")
    kern = functools.partial(
        pl.kernel,
        mesh=mesh,
        out_type=(
            jax.ShapeDtypeStruct((n_rows, k_dim), jnp.float32),
            jax.ShapeDtypeStruct((SC_NW, SC_L), jnp.int32),
        ),
        scratch_types=[
            pltpu.VMEM((d_dim, k_dim), jnp.float32),
            pltpu.VMEM((k_dim,), jnp.float32),
            pltpu.VMEM((nr, d_dim), jnp.float32),
            pltpu.VMEM((SC_L,), jnp.int32),
        ],
    )(functools.partial(_sc_body, d_dim, k_dim, nr, row0))
    return kern(x, ct)


def kernel(x, categories):
    b = x.shape[0]
    k_dim = categories.shape[0]
    tc_rows = b - SC_ROWS
    ct = categories.T
    sc_out, sc_idxp = _sc_call(x, ct, tc_rows, SC_ROWS)
    tc_out, tc_idx = _tc_call(x, categories, tc_rows, b)
    nr = SC_ROWS // SC_NW
    out = lax.dynamic_update_slice(tc_out, sc_out, (tc_rows, 0))
    sc_idx = sc_idxp[:, :nr].reshape(SC_ROWS, 1)
    idx = lax.dynamic_update_slice(tc_idx, sc_idx, (tc_rows, 0))
    return (out, idx.reshape(b))


# speedup vs baseline: 2.0231x; 2.0231x over previous
import jax
import jax.numpy as jnp
from jax import lax
from jax.experimental import pallas as pl
from jax.experimental.pallas import tpu as pltpu

VIGILANCE = 0.75
SUB = 8
ROWS_PER_PROG = 256
LANES = 128


def _combine(s1, j1, s2, j2):
    take2 = (s2 > s1) | ((s2 == s1) & (j2 < j1))
    return jnp.maximum(s1, s2), jnp.where(take2, j2, j1)


def _body(x_ref, c_ref, out_ref, idx_ref, ctb_ref, ps_ref, pj_ref):
    d_dim = x_ref.shape[1]
    k_dim = c_ref.shape[0]
    nsub = ROWS_PER_PROG // SUB
    nchunk = k_dim // LANES

    @pl.when(pl.program_id(0) == 0)
    def _fill():
        ct = jnp.transpose(c_ref[...], (1, 0))
        for d in range(d_dim):
            ctb_ref[d] = jnp.broadcast_to(ct[d:d + 1, :], (SUB, k_dim))

    def sub_block(jb):
        base = jb * SUB
        x = x_ref[pl.ds(base, SUB), :]
        den = jnp.sum(x, axis=1, keepdims=True)
        acc = jnp.zeros((SUB, k_dim), jnp.float32)
        for d in range(d_dim):
            acc = acc + jnp.minimum(x[:, d:d + 1], ctb_ref[d])
        m = acc / den
        s = jnp.where(m >= VIGILANCE, m, jnp.zeros_like(m))
        out_ref[pl.ds(base, SUB), :] = s
        lane = lax.broadcasted_iota(jnp.int32, (SUB, LANES), 1)
        pairs = [(s[:, c * LANES:(c + 1) * LANES], lane + c * LANES)
                 for c in range(nchunk)]
        while len(pairs) > 1:
            nxt = []
            for a in range(0, len(pairs) - 1, 2):
                nxt.append(_combine(*pairs[a], *pairs[a + 1]))
            if len(pairs) % 2:
                nxt.append(pairs[-1])
            pairs = nxt
        ps_ref[jb] = pairs[0][0]
        pj_ref[jb] = pairs[0][1]

    def pair_iter(j, carry):
        sub_block(2 * j)
        sub_block(2 * j + 1)
        return carry

    lax.fori_loop(0, nsub // 2, pair_iter, 0)

    sv = ps_ref[...]
    jv = pj_ref[...]
    t = 1
    while t < LANES:
        sr = pltpu.roll(sv, t, 2)
        jr = pltpu.roll(jv, t, 2)
        sv, jv = _combine(sv, jv, sr, jr)
        t *= 2
    idx_ref[...] = jv[:, :, 0:1].reshape(ROWS_PER_PROG, 1)


def kernel(x, categories):
    b, d_dim = x.shape
    k_dim = categories.shape[0]
    out, idx = pl.pallas_call(
        _body,
        grid=(b // ROWS_PER_PROG,),
        in_specs=[
            pl.BlockSpec((ROWS_PER_PROG, d_dim), lambda i: (i, 0)),
            pl.BlockSpec((k_dim, d_dim), lambda i: (0, 0)),
        ],
        out_specs=[
            pl.BlockSpec((ROWS_PER_PROG, k_dim), lambda i: (i, 0)),
            pl.BlockSpec((ROWS_PER_PROG, 1), lambda i: (i, 0)),
        ],
        out_shape=[
            jax.ShapeDtypeStruct((b, k_dim), jnp.float32),
            jax.ShapeDtypeStruct((b, 1), jnp.int32),
        ],
        scratch_shapes=[
            pltpu.VMEM((d_dim, SUB, k_dim), jnp.float32),
            pltpu.VMEM((ROWS_PER_PROG // SUB, SUB, LANES), jnp.float32),
            pltpu.VMEM((ROWS_PER_PROG // SUB, SUB, LANES), jnp.int32),
        ],
    )(x, categories)
    return (out, idx.reshape(b))


# speedup vs baseline: 2.0317x; 1.0042x over previous
import jax
import jax.numpy as jnp
from jax import lax
from jax.experimental import pallas as pl
from jax.experimental.pallas import tpu as pltpu

VIGILANCE = 0.75
SUB = 8
ROWS_PER_PROG = 512
LANES = 128


def _combine(s1, j1, s2, j2):
    take2 = (s2 > s1) | ((s2 == s1) & (j2 < j1))
    return jnp.maximum(s1, s2), jnp.where(take2, j2, j1)


def _body(x_ref, c_ref, out_ref, idx_ref, ctb_ref, ps_ref, pj_ref):
    d_dim = x_ref.shape[1]
    k_dim = c_ref.shape[0]
    nsub = ROWS_PER_PROG // SUB
    nchunk = k_dim // LANES

    @pl.when(pl.program_id(0) == 0)
    def _fill():
        ct = jnp.transpose(c_ref[...], (1, 0))
        for d in range(d_dim):
            ctb_ref[d] = jnp.broadcast_to(ct[d:d + 1, :], (SUB, k_dim))

    def sub_block(jb):
        base = jb * SUB
        x = x_ref[pl.ds(base, SUB), :]
        den = jnp.sum(x, axis=1, keepdims=True)
        acc = jnp.zeros((SUB, k_dim), jnp.float32)
        for d in range(d_dim):
            acc = acc + jnp.minimum(x[:, d:d + 1], ctb_ref[d])
        m = acc / den
        s = jnp.where(m >= VIGILANCE, m, jnp.zeros_like(m))
        out_ref[pl.ds(base, SUB), :] = s
        lane = lax.broadcasted_iota(jnp.int32, (SUB, LANES), 1)
        pairs = [(s[:, c * LANES:(c + 1) * LANES], lane + c * LANES)
                 for c in range(nchunk)]
        while len(pairs) > 1:
            nxt = []
            for a in range(0, len(pairs) - 1, 2):
                nxt.append(_combine(*pairs[a], *pairs[a + 1]))
            if len(pairs) % 2:
                nxt.append(pairs[-1])
            pairs = nxt
        ps_ref[jb] = pairs[0][0]
        pj_ref[jb] = pairs[0][1]

    def pair_iter(j, carry):
        sub_block(2 * j)
        sub_block(2 * j + 1)
        return carry

    lax.fori_loop(0, nsub // 2, pair_iter, 0)

    sv = ps_ref[...]
    jv = pj_ref[...]
    t = 1
    while t < LANES:
        sr = pltpu.roll(sv, t, 2)
        jr = pltpu.roll(jv, t, 2)
        sv, jv = _combine(sv, jv, sr, jr)
        t *= 2
    idx_ref[...] = jv[:, :, 0:1].reshape(ROWS_PER_PROG, 1)


def kernel(x, categories):
    b, d_dim = x.shape
    k_dim = categories.shape[0]
    out, idx = pl.pallas_call(
        _body,
        grid=(b // ROWS_PER_PROG,),
        in_specs=[
            pl.BlockSpec((ROWS_PER_PROG, d_dim), lambda i: (i, 0)),
            pl.BlockSpec((k_dim, d_dim), lambda i: (0, 0)),
        ],
        out_specs=[
            pl.BlockSpec((ROWS_PER_PROG, k_dim), lambda i: (i, 0)),
            pl.BlockSpec((ROWS_PER_PROG, 1), lambda i: (i, 0)),
        ],
        out_shape=[
            jax.ShapeDtypeStruct((b, k_dim), jnp.float32),
            jax.ShapeDtypeStruct((b, 1), jnp.int32),
        ],
        scratch_shapes=[
            pltpu.VMEM((d_dim, SUB, k_dim), jnp.float32),
            pltpu.VMEM((ROWS_PER_PROG // SUB, SUB, LANES), jnp.float32),
            pltpu.VMEM((ROWS_PER_PROG // SUB, SUB, LANES), jnp.int32),
        ],
    )(x, categories)
    return (out, idx.reshape(b))
